# Initial kernel scaffold; baseline (speedup 1.0000x reference)
import functools

import jax
import jax.numpy as jnp
from jax import lax
from jax.experimental import pallas as pl
from jax.experimental.pallas import tpu as pltpu
from jax.experimental.pallas import tpu_sc as plsc


def _make_sc_kernel(B, S, H, V):
    NC, NS = 2, 16
    NW = NC * NS
    TOTAL = B * S
    RPW = TOTAL // NW
    assert RPW * NW == TOTAL and RPW % S == 0
    NB = 4
    CH = NB * S
    NCH = RPW // CH
    assert NCH * CH == RPW
    SUBS = []
    off = 0
    while off < S:
        sz = min(128, S - off)
        SUBS.append((off, sz))
        off += sz

    mesh = plsc.VectorSubcoreMesh(core_axis_name="c", subcore_axis_name="s")

    @functools.partial(
        pl.kernel,
        mesh=mesh,
        out_type=jax.ShapeDtypeStruct((TOTAL, H), jnp.float32),
        scratch_types=[
            pltpu.VMEM((RPW,), jnp.int32),
            pltpu.VMEM((CH, H), jnp.float32),
            pltpu.VMEM((CH, H), jnp.float32),
            pltpu.SemaphoreType.DMA,
        ],
    )
    def k(x_hbm, emb_hbm, pos_hbm, out_hbm, idx_v, buf, pos_v, gsem):
        wid = lax.axis_index("s") * NC + lax.axis_index("c")
        wbase = pl.multiple_of(wid * RPW, RPW)
        pltpu.sync_copy(x_hbm.at[pl.ds(wbase, RPW)], idx_v)
        for b in range(NB):
            pltpu.sync_copy(pos_hbm, pos_v.at[pl.ds(b * S, S)])

        def chunk_body(c, carry):
            row0 = pl.multiple_of(c * CH, CH)
            descs = []
            for b in range(NB):
                for sub, sz in SUBS:
                    loc = b * S + sub
                    descs.append(
                        pltpu.async_copy(
                            emb_hbm.at[idx_v.at[pl.ds(row0 + loc, sz)]],
                            buf.at[pl.ds(loc, sz)],
                            gsem,
                        )
                    )
            for d in descs:
                d.wait()

            def add_body(r, carry2):
                for hh in range(H // 16):
                    sl = pl.ds(hh * 16, 16)
                    buf[r, sl] = buf[r, sl] + pos_v[r, sl]
                return carry2

            lax.fori_loop(0, CH, add_body, 0, unroll=4)
            pltpu.sync_copy(buf, out_hbm.at[pl.ds(wbase + row0, CH)])
            return carry

        lax.fori_loop(0, NCH, chunk_body, 0)

    return k


def kernel(x, emb, pos_emb):
    B, S = x.shape
    V, H = emb.shape
    xf = x.reshape(B * S).astype(jnp.int32)
    k = _make_sc_kernel(B, S, H, V)
    out = k(xf, emb, pos_emb)
    return out.reshape(B, S, H)


# speedup vs baseline: 1.1891x; 1.1891x over previous
import functools

import jax
import jax.numpy as jnp
from jax import lax
from jax.experimental import pallas as pl
from jax.experimental.pallas import tpu as pltpu
from jax.experimental.pallas import tpu_sc as plsc


def _make_sc_kernel(B, S, H, V):
    NC, NS = 2, 16
    NW = NC * NS
    TOTAL = B * S
    RPW = TOTAL // NW
    assert RPW * NW == TOTAL and RPW % S == 0
    NB = 4
    CH = NB * S
    NCH = RPW // CH
    assert NCH * CH == RPW
    SUBS = []
    off = 0
    while off < S:
        sz = min(128, S - off)
        SUBS.append((off, sz))
        off += sz

    mesh = plsc.VectorSubcoreMesh(core_axis_name="c", subcore_axis_name="s")

    @functools.partial(
        pl.kernel,
        mesh=mesh,
        compiler_params=pltpu.CompilerParams(use_tc_tiling_on_sc=False),
        out_type=jax.ShapeDtypeStruct((TOTAL, H), jnp.float32),
        scratch_types=[
            pltpu.VMEM((RPW,), jnp.int32),
            pltpu.VMEM((CH, H), jnp.float32),
            pltpu.VMEM((CH, H), jnp.float32),
            pltpu.SemaphoreType.DMA,
        ],
    )
    def k(x_hbm, emb_hbm, pos_hbm, out_hbm, idx_v, buf, pos_v, gsem):
        wid = lax.axis_index("s") * NC + lax.axis_index("c")
        wbase = pl.multiple_of(wid * RPW, RPW)
        pltpu.sync_copy(x_hbm.at[pl.ds(wbase, RPW)], idx_v)
        for b in range(NB):
            pltpu.sync_copy(pos_hbm, pos_v.at[pl.ds(b * S, S)])

        def chunk_body(c, carry):
            row0 = pl.multiple_of(c * CH, CH)
            descs = []
            for b in range(NB):
                for sub, sz in SUBS:
                    loc = b * S + sub
                    descs.append(
                        pltpu.async_copy(
                            emb_hbm.at[idx_v.at[pl.ds(row0 + loc, sz)]],
                            buf.at[pl.ds(loc, sz)],
                            gsem,
                        )
                    )
            for d in descs:
                d.wait()

            def add_body(r, carry2):
                for hh in range(H // 16):
                    sl = pl.ds(hh * 16, 16)
                    buf[r, sl] = buf[r, sl] + pos_v[r, sl]
                return carry2

            lax.fori_loop(0, CH, add_body, 0, unroll=4)
            pltpu.sync_copy(buf, out_hbm.at[pl.ds(wbase + row0, CH)])
            return carry

        lax.fori_loop(0, NCH, chunk_body, 0)

    return k


def kernel(x, emb, pos_emb):
    B, S = x.shape
    V, H = emb.shape
    xf = x.reshape(B * S).astype(jnp.int32)
    k = _make_sc_kernel(B, S, H, V)
    out = k(xf, emb, pos_emb)
    return out.reshape(B, S, H)


# speedup vs baseline: 1.2400x; 1.0428x over previous
import functools

import jax
import jax.numpy as jnp
from jax import lax
from jax.experimental import pallas as pl
from jax.experimental.pallas import tpu as pltpu
from jax.experimental.pallas import tpu_sc as plsc


def _make_sc_kernel(B, S, H, V):
    NC, NS = 2, 16
    NW = NC * NS
    TOTAL = B * S
    RPW = TOTAL // NW
    assert RPW * NW == TOTAL and RPW % S == 0
    NB = 4
    CH = NB * S
    NCH = RPW // CH
    assert NCH * CH == RPW and NCH % 2 == 0
    SUBS = []
    off = 0
    while off < S:
        sz = min(128, S - off)
        SUBS.append((off, sz))
        off += sz

    mesh = plsc.VectorSubcoreMesh(core_axis_name="c", subcore_axis_name="s")

    @functools.partial(
        pl.kernel,
        mesh=mesh,
        compiler_params=pltpu.CompilerParams(use_tc_tiling_on_sc=False),
        out_type=jax.ShapeDtypeStruct((TOTAL, H), jnp.float32),
        scratch_types=[
            pltpu.VMEM((RPW,), jnp.int32),
            pltpu.VMEM((CH, H), jnp.float32),
            pltpu.VMEM((CH, H), jnp.float32),
            pltpu.VMEM((CH, H), jnp.float32),
            pltpu.SemaphoreType.DMA,
            pltpu.SemaphoreType.DMA,
        ],
    )
    def k(x_hbm, emb_hbm, pos_hbm, out_hbm, idx_v, buf0, buf1, pos_v, g0, g1):
        bufs = (buf0, buf1)
        gsems = (g0, g1)
        wid = lax.axis_index("s") * NC + lax.axis_index("c")
        wbase = pl.multiple_of(wid * RPW, RPW)
        pltpu.sync_copy(x_hbm.at[pl.ds(wbase, RPW)], idx_v)
        for b in range(NB):
            pltpu.sync_copy(pos_hbm, pos_v.at[pl.ds(b * S, S)])

        def start_gathers(c, buf, gsem):
            row0 = pl.multiple_of(c * CH, CH)
            for b in range(NB):
                for sub, sz in SUBS:
                    loc = b * S + sub
                    pltpu.async_copy(
                        emb_hbm.at[idx_v.at[pl.ds(row0 + loc, sz)]],
                        buf.at[pl.ds(loc, sz)],
                        gsem,
                    )

        def finish_chunk(c, buf, gsem):
            pltpu.make_async_copy(emb_hbm.at[pl.ds(0, CH)], buf, gsem).wait()

            def add_body(r, carry2):
                for hh in range(H // 16):
                    sl = pl.ds(hh * 16, 16)
                    buf[r, sl] = buf[r, sl] + pos_v[r, sl]
                return carry2

            lax.fori_loop(0, CH, add_body, 0, unroll=4)
            row0 = pl.multiple_of(c * CH, CH)
            pltpu.sync_copy(buf, out_hbm.at[pl.ds(wbase + row0, CH)])

        start_gathers(0, buf0, g0)

        def pair_body(p, carry):
            for i in range(2):
                c = p * 2 + i

                @pl.when(c + 1 < NCH)
                def _():
                    start_gathers(c + 1, bufs[1 - i], gsems[1 - i])

                finish_chunk(c, bufs[i], gsems[i])
            return carry

        lax.fori_loop(0, NCH // 2, pair_body, 0)

    return k


def kernel(x, emb, pos_emb):
    B, S = x.shape
    V, H = emb.shape
    xf = x.reshape(B * S).astype(jnp.int32)
    k = _make_sc_kernel(B, S, H, V)
    out = k(xf, emb, pos_emb)
    return out.reshape(B, S, H)
